# baseline (speedup 1.0000x reference)
import jax
import jax.numpy as jnp
from jax.experimental import pallas as pl
from jax.experimental.pallas import tpu as pltpu


def _pick_tile(n, want):
    for t in (want, 1000, 400, 200, 128, 64, 32, 16, 8):
        if t <= want and n % t == 0:
            return t
    return n


def _support_kernel(x_ref, w_ref, s_ref):
    s_ref[...] = jnp.dot(
        x_ref[...].astype(jnp.bfloat16), w_ref[...],
        preferred_element_type=jnp.float32).astype(jnp.bfloat16)


def _first_kernel(s_ref, b_ref, wn_ref, adj_ref, adjb_ref, sn_ref):
    a = adj_ref[...].astype(jnp.bfloat16)
    npad = adjb_ref.shape[1] - a.shape[1]
    if npad:
        a_out = jnp.concatenate(
            [a, jnp.zeros((a.shape[0], npad), jnp.bfloat16)], axis=1)
    else:
        a_out = a
    adjb_ref[...] = a_out
    acc = jnp.dot(a, s_ref[...], preferred_element_type=jnp.float32)
    h = jnp.maximum(acc + b_ref[...], 0.0)
    sn_ref[...] = jnp.dot(
        h.astype(jnp.bfloat16), wn_ref[...],
        preferred_element_type=jnp.float32).astype(jnp.bfloat16)


def _mid_kernel(s_ref, b_ref, wn_ref, adj_ref, sn_ref, acc_ref):
    k = pl.program_id(1)
    nk = pl.num_programs(1)
    part = jnp.dot(adj_ref[...], s_ref[...], preferred_element_type=jnp.float32)

    @pl.when(k == 0)
    def _():
        acc_ref[...] = part

    @pl.when(k > 0)
    def _():
        acc_ref[...] += part

    @pl.when(k == nk - 1)
    def _():
        h = jnp.maximum(acc_ref[...] + b_ref[...], 0.0)
        sn_ref[...] = jnp.dot(
            h.astype(jnp.bfloat16), wn_ref[...],
            preferred_element_type=jnp.float32).astype(jnp.bfloat16)


def _last_kernel(s_ref, b_ref, adj_ref, out_ref, acc_ref):
    k = pl.program_id(1)
    nk = pl.num_programs(1)
    part = jnp.dot(adj_ref[...], s_ref[...], preferred_element_type=jnp.float32)

    @pl.when(k == 0)
    def _():
        acc_ref[...] = part

    @pl.when(k > 0)
    def _():
        acc_ref[...] += part

    @pl.when(k == nk - 1)
    def _():
        out_ref[...] = jnp.maximum(acc_ref[...] + b_ref[...], 0.0)


def _kslice(s, kc):
    npad = kc - s.shape[0]
    if npad:
        s = jnp.pad(s, ((0, npad), (0, 0)))
    return s


def kernel(x, adj, W1, b1, W2, b2, W3, b3, W4, b4):
    n, f = x.shape
    h_dim = W1.shape[1]
    fout = W4.shape[1]
    w2b, w3b, w4b = (w.astype(jnp.bfloat16) for w in (W2, W3, W4))
    npd = -(-n // 256) * 256

    ts = _pick_tile(n, 1000)
    s1 = pl.pallas_call(
        _support_kernel,
        grid=(n // ts,),
        in_specs=[pl.BlockSpec((ts, f), lambda i: (i, 0)),
                  pl.BlockSpec((f, h_dim), lambda i: (0, 0))],
        out_specs=pl.BlockSpec((ts, h_dim), lambda i: (i, 0)),
        out_shape=jax.ShapeDtypeStruct((n, h_dim), jnp.bfloat16),
        compiler_params=pltpu.CompilerParams(
            dimension_semantics=("parallel",)),
    )(x, W1.astype(jnp.bfloat16))

    def resident(arr):
        shp = arr.shape
        return pl.BlockSpec(shp, lambda *_: (0,) * len(shp))

    t1 = _pick_tile(n, 400)
    adjb, s2 = pl.pallas_call(
        _first_kernel,
        grid=(n // t1,),
        in_specs=[resident(s1), resident(b1.reshape(1, h_dim)),
                  resident(w2b),
                  pl.BlockSpec((t1, n), lambda i: (i, 0))],
        out_specs=[pl.BlockSpec((t1, npd), lambda i: (i, 0)),
                   pl.BlockSpec((t1, h_dim), lambda i: (i, 0))],
        out_shape=[jax.ShapeDtypeStruct((n, npd), jnp.bfloat16),
                   jax.ShapeDtypeStruct((n, h_dim), jnp.bfloat16)],
        compiler_params=pltpu.CompilerParams(
            dimension_semantics=("parallel",)),
    )(s1, b1.reshape(1, h_dim), w2b, adj)

    tm = 2000 if (n % 2000 == 0 and npd % 2048 == 0) else _pick_tile(n, 1000)
    kc = 2048 if npd % 2048 == 0 else npd
    nk = npd // kc
    grid = (n // tm, nk)

    def mid(s, b, wn):
        return pl.pallas_call(
            _mid_kernel,
            grid=grid,
            in_specs=[
                pl.BlockSpec((kc, h_dim), lambda i, k: (k, 0)),
                resident(b.reshape(1, h_dim)),
                resident(wn),
                pl.BlockSpec((tm, kc), lambda i, k: (i, k)),
            ],
            out_specs=pl.BlockSpec((tm, h_dim), lambda i, k: (i, 0)),
            out_shape=jax.ShapeDtypeStruct((n, wn.shape[1]), jnp.bfloat16),
            scratch_shapes=[pltpu.VMEM((tm, h_dim), jnp.float32)],
            compiler_params=pltpu.CompilerParams(
                dimension_semantics=("parallel", "arbitrary")),
        )(_kslice(s, npd), b.reshape(1, h_dim), wn, adjb)

    s3 = mid(s2, b2, w3b)
    s4 = mid(s3, b3, w4b)

    x_hat = pl.pallas_call(
        _last_kernel,
        grid=grid,
        in_specs=[
            pl.BlockSpec((kc, h_dim), lambda i, k: (k, 0)),
            resident(b4.reshape(1, fout)),
            pl.BlockSpec((tm, kc), lambda i, k: (i, k)),
        ],
        out_specs=pl.BlockSpec((tm, fout), lambda i, k: (i, 0)),
        out_shape=jax.ShapeDtypeStruct((n, fout), jnp.float32),
        scratch_shapes=[pltpu.VMEM((tm, fout), jnp.float32)],
        compiler_params=pltpu.CompilerParams(
            dimension_semantics=("parallel", "arbitrary")),
    )(_kslice(s4, npd), b4.reshape(1, fout), adjb)
    return x_hat


# speedup vs baseline: 1.1355x; 1.1355x over previous
import jax
import jax.numpy as jnp
from jax.experimental import pallas as pl
from jax.experimental.pallas import tpu as pltpu


def _pick_tile(n, want):
    for t in (want, 1000, 400, 200, 128, 64, 32, 16, 8):
        if t <= want and n % t == 0:
            return t
    return n


def _support_kernel(x_ref, w_ref, s_ref):
    s_ref[...] = jnp.dot(
        x_ref[...].astype(jnp.bfloat16), w_ref[...],
        preferred_element_type=jnp.float32).astype(jnp.bfloat16)


def _first_kernel(s_ref, b_ref, wn_ref, adj_ref, adjb_ref, sn_ref):
    a = adj_ref[...].astype(jnp.bfloat16)
    adjb_ref[...] = a
    acc = jnp.dot(a, s_ref[...], preferred_element_type=jnp.float32)
    h = jnp.maximum(acc + b_ref[...], 0.0)
    sn_ref[...] = jnp.dot(
        h.astype(jnp.bfloat16), wn_ref[...],
        preferred_element_type=jnp.float32).astype(jnp.bfloat16)


def _mid_kernel(s_ref, b_ref, wn_ref, adj_ref, sn_ref):
    acc = jnp.dot(adj_ref[...], s_ref[...], preferred_element_type=jnp.float32)
    h = jnp.maximum(acc + b_ref[...], 0.0)
    sn_ref[...] = jnp.dot(
        h.astype(jnp.bfloat16), wn_ref[...],
        preferred_element_type=jnp.float32).astype(jnp.bfloat16)


def _last_kernel(s_ref, b_ref, adj_ref, out_ref):
    acc = jnp.dot(adj_ref[...], s_ref[...], preferred_element_type=jnp.float32)
    out_ref[...] = jnp.maximum(acc + b_ref[...], 0.0)


def kernel(x, adj, W1, b1, W2, b2, W3, b3, W4, b4):
    n, f = x.shape
    h_dim = W1.shape[1]
    fout = W4.shape[1]
    w2b, w3b, w4b = (w.astype(jnp.bfloat16) for w in (W2, W3, W4))

    ts = _pick_tile(n, 1000)
    s1 = pl.pallas_call(
        _support_kernel,
        grid=(n // ts,),
        in_specs=[pl.BlockSpec((ts, f), lambda i: (i, 0)),
                  pl.BlockSpec((f, h_dim), lambda i: (0, 0))],
        out_specs=pl.BlockSpec((ts, h_dim), lambda i: (i, 0)),
        out_shape=jax.ShapeDtypeStruct((n, h_dim), jnp.bfloat16),
        compiler_params=pltpu.CompilerParams(
            dimension_semantics=("parallel",)),
    )(x, W1.astype(jnp.bfloat16))

    def resident(arr):
        shp = arr.shape
        return pl.BlockSpec(shp, lambda *_: (0,) * len(shp))

    t1 = _pick_tile(n, 200)
    adjb, s2 = pl.pallas_call(
        _first_kernel,
        grid=(n // t1,),
        in_specs=[resident(s1), resident(b1.reshape(1, h_dim)),
                  resident(w2b),
                  pl.BlockSpec((t1, n), lambda i: (i, 0))],
        out_specs=[pl.BlockSpec((t1, n), lambda i: (i, 0)),
                   pl.BlockSpec((t1, h_dim), lambda i: (i, 0))],
        out_shape=[jax.ShapeDtypeStruct((n, n), jnp.bfloat16),
                   jax.ShapeDtypeStruct((n, h_dim), jnp.bfloat16)],
        compiler_params=pltpu.CompilerParams(
            dimension_semantics=("parallel",)),
    )(s1, b1.reshape(1, h_dim), w2b, adj)

    tm = _pick_tile(n, 1000)

    def mid(s, b, wn):
        return pl.pallas_call(
            _mid_kernel,
            grid=(n // tm,),
            in_specs=[resident(s), resident(b.reshape(1, h_dim)),
                      resident(wn),
                      pl.BlockSpec((tm, n), lambda i: (i, 0))],
            out_specs=pl.BlockSpec((tm, h_dim), lambda i: (i, 0)),
            out_shape=jax.ShapeDtypeStruct((n, wn.shape[1]), jnp.bfloat16),
            compiler_params=pltpu.CompilerParams(
                dimension_semantics=("parallel",)),
        )(s, b.reshape(1, h_dim), wn, adjb)

    s3 = mid(s2, b2, w3b)
    s4 = mid(s3, b3, w4b)

    x_hat = pl.pallas_call(
        _last_kernel,
        grid=(n // tm,),
        in_specs=[resident(s4), resident(b4.reshape(1, fout)),
                  pl.BlockSpec((tm, n), lambda i: (i, 0))],
        out_specs=pl.BlockSpec((tm, fout), lambda i: (i, 0)),
        out_shape=jax.ShapeDtypeStruct((n, fout), jnp.float32),
        compiler_params=pltpu.CompilerParams(
            dimension_semantics=("parallel",)),
    )(s4, b4.reshape(1, fout), adjb)
    return x_hat


# speedup vs baseline: 1.1482x; 1.0112x over previous
import jax
import jax.numpy as jnp
from jax.experimental import pallas as pl
from jax.experimental.pallas import tpu as pltpu


def _pick_tile(n, want):
    for t in (want, 1000, 400, 200, 128, 64, 32, 16, 8):
        if t <= want and n % t == 0:
            return t
    return n


def _first_kernel(x_ref, w1_ref, b_ref, wn_ref, adj_ref, adjb_ref, sn_ref,
                  s_ref):
    @pl.when(pl.program_id(0) == 0)
    def _():
        s_ref[...] = jnp.dot(
            x_ref[...].astype(jnp.bfloat16), w1_ref[...],
            preferred_element_type=jnp.float32).astype(jnp.bfloat16)

    a = adj_ref[...].astype(jnp.bfloat16)
    adjb_ref[...] = a
    acc = jnp.dot(a, s_ref[...], preferred_element_type=jnp.float32)
    h = jnp.maximum(acc + b_ref[...], 0.0)
    sn_ref[...] = jnp.dot(
        h.astype(jnp.bfloat16), wn_ref[...],
        preferred_element_type=jnp.float32).astype(jnp.bfloat16)


def _mid_kernel(s_ref, b_ref, wn_ref, adj_ref, sn_ref):
    acc = jnp.dot(adj_ref[...], s_ref[...], preferred_element_type=jnp.float32)
    h = jnp.maximum(acc + b_ref[...], 0.0)
    sn_ref[...] = jnp.dot(
        h.astype(jnp.bfloat16), wn_ref[...],
        preferred_element_type=jnp.float32).astype(jnp.bfloat16)


def _last_kernel(s_ref, b_ref, adj_ref, out_ref):
    acc = jnp.dot(adj_ref[...], s_ref[...], preferred_element_type=jnp.float32)
    out_ref[...] = jnp.maximum(acc + b_ref[...], 0.0)


def kernel(x, adj, W1, b1, W2, b2, W3, b3, W4, b4):
    n, f = x.shape
    h_dim = W1.shape[1]
    fout = W4.shape[1]
    w2b, w3b, w4b = (w.astype(jnp.bfloat16) for w in (W2, W3, W4))

    def resident(arr):
        shp = arr.shape
        return pl.BlockSpec(shp, lambda *_: (0,) * len(shp))

    t1 = _pick_tile(n, 400)
    adjb, s2 = pl.pallas_call(
        _first_kernel,
        grid=(n // t1,),
        in_specs=[resident(x), resident(W1.astype(jnp.bfloat16)),
                  resident(b1.reshape(1, h_dim)),
                  resident(w2b),
                  pl.BlockSpec((t1, n), lambda i: (i, 0))],
        out_specs=[pl.BlockSpec((t1, n), lambda i: (i, 0)),
                   pl.BlockSpec((t1, h_dim), lambda i: (i, 0))],
        out_shape=[jax.ShapeDtypeStruct((n, n), jnp.bfloat16),
                   jax.ShapeDtypeStruct((n, h_dim), jnp.bfloat16)],
        scratch_shapes=[pltpu.VMEM((n, h_dim), jnp.bfloat16)],
        compiler_params=pltpu.CompilerParams(
            dimension_semantics=("arbitrary",),
            vmem_limit_bytes=64 * 1024 * 1024),
    )(x, W1.astype(jnp.bfloat16), b1.reshape(1, h_dim), w2b, adj)

    tm = _pick_tile(n, 1000)

    def mid(s, b, wn):
        return pl.pallas_call(
            _mid_kernel,
            grid=(n // tm,),
            in_specs=[resident(s), resident(b.reshape(1, h_dim)),
                      resident(wn),
                      pl.BlockSpec((tm, n), lambda i: (i, 0))],
            out_specs=pl.BlockSpec((tm, h_dim), lambda i: (i, 0)),
            out_shape=jax.ShapeDtypeStruct((n, wn.shape[1]), jnp.bfloat16),
            compiler_params=pltpu.CompilerParams(
                dimension_semantics=("parallel",)),
        )(s, b.reshape(1, h_dim), wn, adjb)

    s3 = mid(s2, b2, w3b)
    s4 = mid(s3, b3, w4b)

    x_hat = pl.pallas_call(
        _last_kernel,
        grid=(n // tm,),
        in_specs=[resident(s4), resident(b4.reshape(1, fout)),
                  pl.BlockSpec((tm, n), lambda i: (i, 0))],
        out_specs=pl.BlockSpec((tm, fout), lambda i: (i, 0)),
        out_shape=jax.ShapeDtypeStruct((n, fout), jnp.float32),
        compiler_params=pltpu.CompilerParams(
            dimension_semantics=("parallel",)),
    )(s4, b4.reshape(1, fout), adjb)
    return x_hat


# speedup vs baseline: 1.1650x; 1.0146x over previous
import jax
import jax.numpy as jnp
from jax.experimental import pallas as pl
from jax.experimental.pallas import tpu as pltpu


def _pick_tile(n, want):
    for t in (want, 1000, 400, 200, 128, 64, 32, 16, 8):
        if t <= want and n % t == 0:
            return t
    return n


def _first_kernel(x_ref, w1_ref, b_ref, wn_ref, adj_ref, adjb_ref, sn_ref,
                  s_ref):
    @pl.when(pl.program_id(0) == 0)
    def _():
        s_ref[...] = jnp.dot(
            x_ref[...].astype(jnp.bfloat16), w1_ref[...].astype(jnp.bfloat16),
            preferred_element_type=jnp.float32).astype(jnp.bfloat16)

    a = adj_ref[...].astype(jnp.bfloat16)
    adjb_ref[...] = a
    acc = jnp.dot(a, s_ref[...], preferred_element_type=jnp.float32)
    h = jnp.maximum(acc + b_ref[...], 0.0)
    sn_ref[...] = jnp.dot(
        h.astype(jnp.bfloat16), wn_ref[...].astype(jnp.bfloat16),
        preferred_element_type=jnp.float32).astype(jnp.bfloat16)


def _mid_kernel(s_ref, b_ref, wn_ref, adj_ref, sn_ref):
    acc = jnp.dot(adj_ref[...], s_ref[...], preferred_element_type=jnp.float32)
    h = jnp.maximum(acc + b_ref[...], 0.0)
    sn_ref[...] = jnp.dot(
        h.astype(jnp.bfloat16), wn_ref[...].astype(jnp.bfloat16),
        preferred_element_type=jnp.float32).astype(jnp.bfloat16)


def _last_kernel(s_ref, b_ref, adj_ref, out_ref):
    acc = jnp.dot(adj_ref[...], s_ref[...], preferred_element_type=jnp.float32)
    out_ref[...] = jnp.maximum(acc + b_ref[...], 0.0)


def kernel(x, adj, W1, b1, W2, b2, W3, b3, W4, b4):
    n, f = x.shape
    h_dim = W1.shape[1]
    fout = W4.shape[1]
    def resident(arr):
        shp = arr.shape
        return pl.BlockSpec(shp, lambda *_: (0,) * len(shp))

    t1 = _pick_tile(n, 400)
    adjb, s2 = pl.pallas_call(
        _first_kernel,
        grid=(n // t1,),
        in_specs=[resident(x), resident(W1),
                  resident(b1.reshape(1, h_dim)),
                  resident(W2),
                  pl.BlockSpec((t1, n), lambda i: (i, 0))],
        out_specs=[pl.BlockSpec((t1, n), lambda i: (i, 0)),
                   pl.BlockSpec((t1, h_dim), lambda i: (i, 0))],
        out_shape=[jax.ShapeDtypeStruct((n, n), jnp.bfloat16),
                   jax.ShapeDtypeStruct((n, h_dim), jnp.bfloat16)],
        scratch_shapes=[pltpu.VMEM((n, h_dim), jnp.bfloat16)],
        compiler_params=pltpu.CompilerParams(
            dimension_semantics=("arbitrary",),
            vmem_limit_bytes=64 * 1024 * 1024),
    )(x, W1, b1.reshape(1, h_dim), W2, adj)

    tm = _pick_tile(n, 1000)

    def mid(s, b, wn):
        return pl.pallas_call(
            _mid_kernel,
            grid=(n // tm,),
            in_specs=[resident(s), resident(b.reshape(1, h_dim)),
                      resident(wn),
                      pl.BlockSpec((tm, n), lambda i: (i, 0))],
            out_specs=pl.BlockSpec((tm, h_dim), lambda i: (i, 0)),
            out_shape=jax.ShapeDtypeStruct((n, wn.shape[1]), jnp.bfloat16),
            compiler_params=pltpu.CompilerParams(
                dimension_semantics=("parallel",)),
        )(s, b.reshape(1, h_dim), wn, adjb)

    s3 = mid(s2, b2, W3)
    s4 = mid(s3, b3, W4)

    x_hat = pl.pallas_call(
        _last_kernel,
        grid=(n // tm,),
        in_specs=[resident(s4), resident(b4.reshape(1, fout)),
                  pl.BlockSpec((tm, n), lambda i: (i, 0))],
        out_specs=pl.BlockSpec((tm, fout), lambda i: (i, 0)),
        out_shape=jax.ShapeDtypeStruct((n, fout), jnp.float32),
        compiler_params=pltpu.CompilerParams(
            dimension_semantics=("parallel",)),
    )(s4, b4.reshape(1, fout), adjb)
    return x_hat
